# Initial kernel scaffold; baseline (speedup 1.0000x reference)
import jax
import jax.numpy as jnp
from jax.experimental import pallas as pl


def kernel(features, labels, prototypes, prototype_counts, initialized):
    raise NotImplementedError("write your pallas kernel here")


# speedup vs baseline: 2.3081x; 2.3081x over previous
import jax
import jax.numpy as jnp
from jax.experimental import pallas as pl
from jax.experimental.pallas import tpu as pltpu

_NUM_CLASSES = 1000
_FEAT_DIM = 512
_TAU = 1.0
_GAMMA = 0.1
_MOMENTUM = 0.999


def _body(feat_ref, lab_ref, proto_ref, init_ref,
          loss_ref, thr_ref, mean_ref, min_ref, max_ref,
          acc_s, acc_c, acc_q):
    C = _NUM_CLASSES
    i = pl.program_id(0)
    nblk = pl.num_programs(0)

    lab = lab_ref[0]
    feats = feat_ref[...]
    class_iota = jax.lax.broadcasted_iota(jnp.int32, (C, lab.shape[1]), 0)
    oh = (class_iota == lab).astype(jnp.float32)
    sums_blk = jax.lax.dot_general(
        oh, feats, (((1,), (0,)), ((), ())),
        precision=jax.lax.Precision.HIGHEST,
        preferred_element_type=jnp.float32)
    rowsq = jnp.sum(feats * feats, axis=1)
    cnt_blk = jnp.sum(oh, axis=1, keepdims=True)
    sq_blk = jnp.sum(oh * rowsq[None, :], axis=1, keepdims=True)

    @pl.when(i == 0)
    def _():
        acc_s[...] = sums_blk
        acc_c[...] = cnt_blk
        acc_q[...] = sq_blk

    @pl.when(i > 0)
    def _():
        acc_s[...] += sums_blk
        acc_c[...] += cnt_blk
        acc_q[...] += sq_blk

    @pl.when(i == nblk - 1)
    def _():
        counts = acc_c[...]
        sums = acc_s[...]
        sqsum = acc_q[...]
        initb = init_ref[...] > 0.5
        active = counts > 0.0
        means = sums / jnp.maximum(counts, 1.0)
        protos = jnp.where(active & (~initb), means, proto_ref[...])
        protos = jnp.where(active & initb,
                           _MOMENTUM * protos + (1.0 - _MOMENTUM) * means,
                           protos)
        init_new = initb | active
        init_new_f = init_new.astype(jnp.float32)

        gram = jax.lax.dot_general(
            protos, protos, (((1,), (1,)), ((), ())),
            precision=jax.lax.Precision.HIGHEST,
            preferred_element_type=jnp.float32)
        eye = (jax.lax.broadcasted_iota(jnp.int32, (C, C), 0)
               == jax.lax.broadcasted_iota(jnp.int32, (C, C), 1))
        gd = jnp.where(eye, gram, 0.0)
        sqcol = jnp.sum(gd, axis=1, keepdims=True)
        sqrow = jnp.sum(gd, axis=0, keepdims=True)
        d2 = jnp.maximum(sqcol + sqrow - 2.0 * gram, 0.0)

        init_row = jnp.sum(
            jnp.where(eye, jnp.broadcast_to(init_new_f, (C, C)), 0.0),
            axis=0, keepdims=True)
        pair = (init_new_f * init_row) > 0.5
        dist = jnp.sqrt(d2)
        dist = jnp.where(pair & (~eye), dist, jnp.inf)
        min_dist = jnp.min(dist)
        n_init = jnp.sum(init_new_f)
        threshold = jnp.where(n_init < 2.0, _TAU, _GAMMA * (min_dist * min_dist))

        pdots = jnp.sum(protos * sums, axis=1, keepdims=True)
        class_sums = sqsum - 2.0 * pdots + counts * sqcol
        spreads = class_sums / jnp.maximum(counts, 1.0)
        valid = counts >= 2.0
        n_valid = jnp.sum(valid.astype(jnp.float32))
        per_class = jnp.maximum(threshold - spreads, 0.0)
        loss = jnp.sum(jnp.where(valid, per_class, 0.0)) / jnp.maximum(n_valid, 1.0)
        loss = jnp.where(n_valid > 0.0, loss, 0.0)
        mean_spread = jnp.sum(jnp.where(valid, spreads, 0.0)) / jnp.maximum(n_valid, 1.0)
        min_spread = jnp.min(jnp.where(valid, spreads, jnp.inf))
        max_spread = jnp.max(jnp.where(valid, spreads, -jnp.inf))

        loss_ref[...] = jnp.broadcast_to(loss, (1, 1))
        thr_ref[...] = jnp.broadcast_to(threshold, (1, 1))
        mean_ref[...] = jnp.broadcast_to(mean_spread, (1, 1))
        min_ref[...] = jnp.broadcast_to(min_spread, (1, 1))
        max_ref[...] = jnp.broadcast_to(max_spread, (1, 1))


def kernel(features, labels, prototypes, prototype_counts, initialized):
    del prototype_counts
    S, D = features.shape
    C = prototypes.shape[0]
    CHUNK = 2048
    G = S // CHUNK
    labels3 = labels.astype(jnp.int32).reshape(G, 1, CHUNK)
    init_col = initialized.astype(jnp.float32).reshape(C, 1)

    outs = pl.pallas_call(
        _body,
        grid=(G,),
        in_specs=[
            pl.BlockSpec((CHUNK, D), lambda i: (i, 0)),
            pl.BlockSpec((1, 1, CHUNK), lambda i: (i, 0, 0)),
            pl.BlockSpec((C, D), lambda i: (0, 0)),
            pl.BlockSpec((C, 1), lambda i: (0, 0)),
        ],
        out_specs=[pl.BlockSpec((1, 1), lambda i: (0, 0))] * 5,
        out_shape=[jax.ShapeDtypeStruct((1, 1), jnp.float32)] * 5,
        scratch_shapes=[
            pltpu.VMEM((C, D), jnp.float32),
            pltpu.VMEM((C, 1), jnp.float32),
            pltpu.VMEM((C, 1), jnp.float32),
        ],
    )(features, labels3, prototypes, init_col)
    loss, thr, mean_s, min_s, max_s = [o[0, 0] for o in outs]
    return loss, thr, mean_s, min_s, max_s
